# baseline (speedup 1.0000x reference)
import jax
import jax.numpy as jnp
from jax.experimental import pallas as pl
from jax.experimental.pallas import tpu as pltpu

_K = 16
_BB = 2


def _bdot(a, b):
    return jnp.dot(a.astype(jnp.bfloat16), b.astype(jnp.bfloat16),
                   preferred_element_type=jnp.float32)


def _pt_layer_kernel(x_ref, wq_ref, wk_ref, wv_ref, wg_ref, wo_ref,
                     bq_ref, bk_ref, bv_ref, bg_ref, bo_ref, out_ref):
    C = wq_ref.shape[0]
    ii = jax.lax.broadcasted_iota(jnp.int32, (C, C), 0)
    jj = jax.lax.broadcasted_iota(jnp.int32, (C, C), 1)
    eye = (ii == jj).astype(jnp.float32)

    def _col(b_ref):
        return jnp.sum(eye * b_ref[...][None, :], axis=1, keepdims=True)

    wqk = (wq_ref[...] - wk_ref[...]).astype(jnp.bfloat16)
    bqk = _col(bq_ref) - _col(bk_ref)
    wo = wo_ref[...]
    wog = wo + jnp.dot(wo, wg_ref[...], preferred_element_type=jnp.float32)
    wog = wog.astype(jnp.bfloat16)
    bog = jnp.dot(wo, _col(bg_ref), preferred_element_type=jnp.float32)
    bog = bog + _col(bo_ref)
    wv = wv_ref[...].astype(jnp.bfloat16)
    bv = _col(bv_ref)
    for i in range(x_ref.shape[0]):
        xb = x_ref[i].astype(jnp.bfloat16)
        s = _bdot(wqk, xb) + bqk
        m = jnp.max(s, axis=0, keepdims=True)
        e = jnp.exp(s - m)
        attn = e / jnp.sum(e, axis=0, keepdims=True)
        v = _bdot(wv, xb) + bv
        xa = (float(_K) * attn) * v
        out = _bdot(wog, xa)
        out_ref[i] = out + bog


@jax.jit
def kernel(x, pos, Wq, bq, Wk, bk, Wv, bv, Wg, bg, Wo, bo):
    del pos
    B, C_in, N = x.shape
    C_out = Wq.shape[0]

    bb = _BB if B % _BB == 0 else B
    tn = N // 2
    grid = (B // bb, 2)

    wspec = pl.BlockSpec((C_out, C_in), lambda b, j: (0, 0))
    bspec = pl.BlockSpec((C_out,), lambda b, j: (0,))

    out = pl.pallas_call(
        _pt_layer_kernel,
        grid=grid,
        in_specs=[
            pl.BlockSpec((bb, C_in, tn), lambda b, j: (b, 0, j)),
            wspec, wspec, wspec, wspec, wspec,
            bspec, bspec, bspec, bspec, bspec,
        ],
        out_specs=pl.BlockSpec((bb, C_out, tn), lambda b, j: (b, 0, j)),
        out_shape=jax.ShapeDtypeStruct((B, C_out, N), jnp.float32),
        compiler_params=pltpu.CompilerParams(
            dimension_semantics=("parallel", "parallel")),
    )(x, Wq, Wk, Wv, Wg, Wo, bq, bk, bv, bg, bo)
    return out


# speedup vs baseline: 1.1328x; 1.1328x over previous
import jax
import jax.numpy as jnp
from jax.experimental import pallas as pl
from jax.experimental.pallas import tpu as pltpu

_K = 16
_BB = 2


def _bdot(a, b):
    return jnp.dot(a.astype(jnp.bfloat16), b.astype(jnp.bfloat16),
                   preferred_element_type=jnp.float32)


def _pt_layer_kernel(x_ref, wq_ref, wk_ref, wv_ref, wg_ref, wo_ref,
                     bq_ref, bk_ref, bv_ref, bg_ref, bo_ref, out_ref):
    C = wq_ref.shape[0]
    ii = jax.lax.broadcasted_iota(jnp.int32, (C, C), 0)
    jj = jax.lax.broadcasted_iota(jnp.int32, (C, C), 1)
    eye = (ii == jj).astype(jnp.float32)

    def _col(b_ref):
        return jnp.sum(eye * b_ref[...][None, :], axis=1, keepdims=True)

    wqk = (wq_ref[...] - wk_ref[...]).astype(jnp.bfloat16)
    bqk = _col(bq_ref) - _col(bk_ref)
    wo = wo_ref[...]
    wog = wo + jnp.dot(wo, wg_ref[...], preferred_element_type=jnp.float32)
    wog = wog.astype(jnp.bfloat16)
    bog = jnp.dot(wo, _col(bg_ref), preferred_element_type=jnp.float32)
    bog = bog + _col(bo_ref)
    wv = wv_ref[...].astype(jnp.bfloat16)
    bv = _col(bv_ref)
    for i in range(x_ref.shape[0]):
        xb = x_ref[i].astype(jnp.bfloat16)
        s = _bdot(wqk, xb) + bqk
        m = jnp.max(s, axis=0, keepdims=True)
        e = jnp.exp(s - m)
        attn = e / jnp.sum(e, axis=0, keepdims=True)
        v = _bdot(wv, xb) + bv
        xa = (float(_K) * attn) * v
        out = _bdot(wog, xa)
        out_ref[i] = out + bog


@jax.jit
def kernel(x, pos, Wq, bq, Wk, bk, Wv, bv, Wg, bg, Wo, bo):
    del pos
    B, C_in, N = x.shape
    C_out = Wq.shape[0]

    bb = _BB if B % _BB == 0 else B
    grid = (B // bb,)

    wspec = pl.BlockSpec((C_out, C_in), lambda b: (0, 0))
    bspec = pl.BlockSpec((C_out,), lambda b: (0,))

    out = pl.pallas_call(
        _pt_layer_kernel,
        grid=grid,
        in_specs=[
            pl.BlockSpec((bb, C_in, N), lambda b: (b, 0, 0)),
            wspec, wspec, wspec, wspec, wspec,
            bspec, bspec, bspec, bspec, bspec,
        ],
        out_specs=pl.BlockSpec((bb, C_out, N), lambda b: (b, 0, 0)),
        out_shape=jax.ShapeDtypeStruct((B, C_out, N), jnp.float32),
        compiler_params=pltpu.CompilerParams(
            dimension_semantics=("parallel",)),
    )(x, Wq, Wk, Wv, Wg, Wo, bq, bk, bv, bg, bo)
    return out
